# baseline (speedup 1.0000x reference)
import functools

import jax
import jax.numpy as jnp
from jax import lax
from jax.experimental import pallas as pl
from jax.experimental.pallas import tpu as pltpu
from jax.experimental.pallas import tpu_sc as plsc

B, L, V, H = 16384, 32, 40960, 256
NC, NS = 2, 16
NW = NC * NS
HV = H // 16
SPB = 2
ROWS = SPB * L
NBUF = 4
OB = 64
QPG = OB // (SPB * NBUF)


def _embed_bags(wf, bf, W_white, W_black, NB):
    SUB = NB // NW
    NSTR = SUB // SPB
    NQ = NSTR // NBUF
    mesh = plsc.VectorSubcoreMesh(core_axis_name="c", subcore_axis_name="s",
                                  num_cores=NC, num_subcores=NS)

    @functools.partial(
        pl.kernel,
        out_type=(
            jax.ShapeDtypeStruct((NB, H), jnp.float32),
            jax.ShapeDtypeStruct((NB, H), jnp.float32),
        ),
        mesh=mesh,
        scratch_types=[
            pltpu.VMEM((SUB * L,), jnp.int32),
            pltpu.VMEM((NBUF, ROWS, H), jnp.float32),
            pltpu.VMEM((OB, H), jnp.float32),
            pltpu.SemaphoreType.DMA,
            pltpu.SemaphoreType.DMA,
            pltpu.SemaphoreType.DMA,
            pltpu.SemaphoreType.DMA,
        ],
    )
    def body(wf_h, bf_h, Ww_h, Wb_h, ow_h, ob_h, idxf, ring, outb, *sg):
        wid = lax.axis_index("s") * NC + lax.axis_index("c")
        zeros = jnp.zeros((16,), jnp.float32)

        def reduce_bag(buf, roff, orow):
            @pl.loop(0, L, init_carry=(zeros,) * HV)
            def acc(l, carry):
                return tuple(
                    carry[j] + buf[roff + l, pl.ds(16 * j, 16)]
                    for j in range(HV))
            for j in range(HV):
                outb[orow, pl.ds(16 * j, 16)] = acc[j]

        def fire(W_h, q, b):
            pltpu.async_copy(
                W_h.at[idxf.at[pl.ds((NBUF * q + b) * ROWS, ROWS)]],
                ring.at[b], sg[b])

        def do_table(W_h, f_h, o_h):
            pltpu.sync_copy(f_h.at[pl.ds(wid * SUB * L, SUB * L)], idxf)
            for b in range(NBUF):
                fire(W_h, 0, b)

            def quad(q, last):
                for b in range(NBUF):
                    pltpu.make_async_copy(
                        W_h.at[idxf.at[pl.ds(0, ROWS)]],
                        ring.at[b], sg[b]).wait()
                    for ss in range(SPB):
                        orow = lax.rem(NBUF * SPB * q + SPB * b + ss,
                                       jnp.int32(OB))
                        reduce_bag(ring.at[b], 32 * ss, orow)
                    if not last:
                        fire(W_h, q + 1, b)

            @pl.loop(0, NQ - 1)
            def _mid(q):
                quad(q, False)

                @pl.when(lax.rem(q, jnp.int32(QPG)) == QPG - 1)
                def _():
                    gbase = wid * SUB + (q // QPG) * OB
                    pltpu.sync_copy(outb, o_h.at[pl.ds(gbase, OB)])

            quad(NQ - 1, True)
            pltpu.sync_copy(outb, o_h.at[pl.ds(wid * SUB + SUB - OB, OB)])

        do_table(Ww_h, wf_h, ow_h)
        do_table(Wb_h, bf_h, ob_h)

    return body(wf, bf, W_white, W_black)


BM = 2048


def _mlp_body(wv, bv, w1a, w1b, b1, w2t, b2, w3r, b3, o):
    f32 = jnp.float32
    hi = jax.lax.Precision.HIGHEST
    h = jnp.dot(wv[...], w1a[...], precision=hi, preferred_element_type=f32)
    h += jnp.dot(bv[...], w1b[...], precision=hi, preferred_element_type=f32)
    h = jnp.maximum(h + b1[...], 0.0)
    h = jnp.dot(h, w2t[...], precision=hi, preferred_element_type=f32)
    h = jnp.maximum(h + b2[...], 0.0)
    o[...] = jnp.sum(h * w3r[...], axis=1) + b3[0, 0]


def _mlp(white_vec, black_vec, w1, b1, w2, b2, w3, b3):
    NB = white_vec.shape[0]
    w1a = w1[:, :H].T
    w1b = w1[:, H:].T
    grid = (NB // BM,)
    full = lambda shape: pl.BlockSpec(shape, lambda i: (0, 0))
    return pl.pallas_call(
        _mlp_body,
        grid=grid,
        in_specs=[
            pl.BlockSpec((BM, H), lambda i: (i, 0)),
            pl.BlockSpec((BM, H), lambda i: (i, 0)),
            full((H, 32)),
            full((H, 32)),
            full((1, 32)),
            full((32, 32)),
            full((1, 32)),
            full((1, 32)),
            full((1, 1)),
        ],
        out_specs=pl.BlockSpec((BM,), lambda i: (i,)),
        out_shape=jax.ShapeDtypeStruct((NB,), jnp.float32),
    )(white_vec, black_vec, w1a, w1b, b1.reshape(1, 32), w2.T,
      b2.reshape(1, 32), w3.reshape(1, 32), b3.reshape(1, 1))


def kernel(white_input, black_input, W_white, W_black, w1, b1, w2, b2, w3, b3):
    wf = white_input.reshape(-1).astype(jnp.int32)
    bf = black_input.reshape(-1).astype(jnp.int32)
    NB = B // 4
    outs = []
    for h in range(4):
        wv, bv = _embed_bags(wf[h * NB * L:(h + 1) * NB * L],
                             bf[h * NB * L:(h + 1) * NB * L],
                             W_white, W_black, NB)
        outs.append(_mlp(wv, bv, w1, b1, w2, b2, w3, b3))
    return jnp.concatenate(outs)


# speedup vs baseline: 1.0562x; 1.0562x over previous
import functools

import jax
import jax.numpy as jnp
from jax import lax
from jax.experimental import pallas as pl
from jax.experimental.pallas import tpu as pltpu
from jax.experimental.pallas import tpu_sc as plsc

B, L, V, H = 16384, 32, 40960, 256
NC, NS = 2, 16
NW = NC * NS
HV = H // 16
SPB = 1
ROWS = SPB * L
NBUF = 8
OB = 64
QPG = OB // (SPB * NBUF)


def _embed_bags(wf, bf, W_white, W_black, NB):
    SUB = NB // NW
    NSTR = SUB // SPB
    NQ = NSTR // NBUF
    mesh = plsc.VectorSubcoreMesh(core_axis_name="c", subcore_axis_name="s",
                                  num_cores=NC, num_subcores=NS)

    @functools.partial(
        pl.kernel,
        out_type=(
            jax.ShapeDtypeStruct((NB, H), jnp.float32),
            jax.ShapeDtypeStruct((NB, H), jnp.float32),
        ),
        mesh=mesh,
        scratch_types=[
            pltpu.VMEM((SUB * L,), jnp.int32),
            pltpu.VMEM((NBUF, ROWS, H), jnp.float32),
            pltpu.VMEM((OB, H), jnp.float32),
        ] + [pltpu.SemaphoreType.DMA] * NBUF,
    )
    def body(wf_h, bf_h, Ww_h, Wb_h, ow_h, ob_h, idxf, ring, outb, *sg):
        wid = lax.axis_index("s") * NC + lax.axis_index("c")
        zeros = jnp.zeros((16,), jnp.float32)

        def reduce_bag(buf, roff, orow):
            @pl.loop(0, L, init_carry=(zeros,) * HV)
            def acc(l, carry):
                return tuple(
                    carry[j] + buf[roff + l, pl.ds(16 * j, 16)]
                    for j in range(HV))
            for j in range(HV):
                outb[orow, pl.ds(16 * j, 16)] = acc[j]

        def fire(W_h, q, b):
            pltpu.async_copy(
                W_h.at[idxf.at[pl.ds((NBUF * q + b) * ROWS, ROWS)]],
                ring.at[b], sg[b])

        def do_table(W_h, f_h, o_h):
            pltpu.sync_copy(f_h.at[pl.ds(wid * SUB * L, SUB * L)], idxf)
            for b in range(NBUF):
                fire(W_h, 0, b)

            def quad(q, last):
                for b in range(NBUF):
                    pltpu.make_async_copy(
                        W_h.at[idxf.at[pl.ds(0, ROWS)]],
                        ring.at[b], sg[b]).wait()
                    for ss in range(SPB):
                        orow = lax.rem(NBUF * SPB * q + SPB * b + ss,
                                       jnp.int32(OB))
                        reduce_bag(ring.at[b], 32 * ss, orow)
                    if not last:
                        fire(W_h, q + 1, b)

            @pl.loop(0, NQ - 1)
            def _mid(q):
                quad(q, False)

                @pl.when(lax.rem(q, jnp.int32(QPG)) == QPG - 1)
                def _():
                    gbase = wid * SUB + (q // QPG) * OB
                    pltpu.sync_copy(outb, o_h.at[pl.ds(gbase, OB)])

            quad(NQ - 1, True)
            pltpu.sync_copy(outb, o_h.at[pl.ds(wid * SUB + SUB - OB, OB)])

        do_table(Ww_h, wf_h, ow_h)
        do_table(Wb_h, bf_h, ob_h)

    return body(wf, bf, W_white, W_black)


BM = 2048


def _mlp_body(wv, bv, w1a, w1b, b1, w2t, b2, w3r, b3, o):
    f32 = jnp.float32
    hi = jax.lax.Precision.HIGHEST
    h = jnp.dot(wv[...], w1a[...], precision=hi, preferred_element_type=f32)
    h += jnp.dot(bv[...], w1b[...], precision=hi, preferred_element_type=f32)
    h = jnp.maximum(h + b1[...], 0.0)
    h = jnp.dot(h, w2t[...], precision=hi, preferred_element_type=f32)
    h = jnp.maximum(h + b2[...], 0.0)
    o[...] = jnp.sum(h * w3r[...], axis=1) + b3[0, 0]


def _mlp(white_vec, black_vec, w1, b1, w2, b2, w3, b3):
    NB = white_vec.shape[0]
    w1a = w1[:, :H].T
    w1b = w1[:, H:].T
    grid = (NB // BM,)
    full = lambda shape: pl.BlockSpec(shape, lambda i: (0, 0))
    return pl.pallas_call(
        _mlp_body,
        grid=grid,
        in_specs=[
            pl.BlockSpec((BM, H), lambda i: (i, 0)),
            pl.BlockSpec((BM, H), lambda i: (i, 0)),
            full((H, 32)),
            full((H, 32)),
            full((1, 32)),
            full((32, 32)),
            full((1, 32)),
            full((1, 32)),
            full((1, 1)),
        ],
        out_specs=pl.BlockSpec((BM,), lambda i: (i,)),
        out_shape=jax.ShapeDtypeStruct((NB,), jnp.float32),
    )(white_vec, black_vec, w1a, w1b, b1.reshape(1, 32), w2.T,
      b2.reshape(1, 32), w3.reshape(1, 32), b3.reshape(1, 1))


def kernel(white_input, black_input, W_white, W_black, w1, b1, w2, b2, w3, b3):
    wf = white_input.reshape(-1).astype(jnp.int32)
    bf = black_input.reshape(-1).astype(jnp.int32)
    NB = B // 2
    outs = []
    for h in range(2):
        wv, bv = _embed_bags(wf[h * NB * L:(h + 1) * NB * L],
                             bf[h * NB * L:(h + 1) * NB * L],
                             W_white, W_black, NB)
        outs.append(_mlp(wv, bv, w1, b1, w2, b2, w3, b3))
    return jnp.concatenate(outs)
